# Initial kernel scaffold; baseline (speedup 1.0000x reference)
import jax
import jax.numpy as jnp
from jax.experimental import pallas as pl


def kernel(slab, wcombo, mask, wa, bias, x):
    raise NotImplementedError("write your pallas kernel here")


# speedup vs baseline: 1.0274x; 1.0274x over previous
import functools

import jax
import jax.numpy as jnp
from jax import lax
from jax.experimental import pallas as pl
from jax.experimental.pallas import tpu as pltpu

_L = 16
_NKH = 896
_HP = 128
_S = 384
_NCOL = _L * _NKH + _HP
_V_BLK = 512
_T_PER_CHUNK = 2


def _fused_body(slab_ref, wcombo_ref, mask_ref, wa_ref, b_ref, x_ref, out_ref):
    slab = slab_ref[...]
    mask = mask_ref[...]

    cw = _T_PER_CHUNK * _NKH
    pooled = None
    for c in range(_L // _T_PER_CHUNK):
        p = jnp.dot(slab, wcombo_ref[:, c * cw:(c + 1) * cw],
                    preferred_element_type=jnp.float32)
        for i in range(_T_PER_CHUNK):
            t = c * _T_PER_CHUNK + i
            cand = p[:, i * _NKH:(i + 1) * _NKH] + mask[t:t + 1, :]
            pooled = cand if pooled is None else jnp.maximum(pooled, cand)

    feat = jnp.tanh(pooled).astype(jnp.bfloat16)
    wproj = jnp.dot(slab, wcombo_ref[:, _L * _NKH:],
                    preferred_element_type=jnp.float32)

    y = jnp.tanh(jnp.dot(feat, wa_ref[...], preferred_element_type=jnp.float32)
                 + wproj + b_ref[...])

    out_ref[...] = lax.dot_general(x_ref[...], y, (((1,), (1,)), ((), ())),
                                   preferred_element_type=jnp.float32)


def kernel(slab, wcombo, mask, wa, bias, x):
    Vp = slab.shape[0]
    B = x.shape[0]
    n_blk = Vp // _V_BLK
    assert n_blk * _V_BLK == Vp

    x32 = x.astype(jnp.float32)

    out = pl.pallas_call(
        _fused_body,
        out_shape=jax.ShapeDtypeStruct((B, Vp), jnp.float32),
        grid=(n_blk,),
        in_specs=[
            pl.BlockSpec((_V_BLK, _S), lambda j: (j, 0)),
            pl.BlockSpec((_S, _NCOL), lambda j: (0, 0)),
            pl.BlockSpec((_L, _NKH), lambda j: (0, 0)),
            pl.BlockSpec((_NKH, _HP), lambda j: (0, 0)),
            pl.BlockSpec((1, _HP), lambda j: (0, 0)),
            pl.BlockSpec((B, _HP), lambda j: (0, 0)),
        ],
        out_specs=pl.BlockSpec((B, _V_BLK), lambda j: (0, j)),
        compiler_params=pltpu.CompilerParams(
            dimension_semantics=("parallel",),
            vmem_limit_bytes=50 * 1024 * 1024),
    )(slab, wcombo, mask, wa, bias, x32)
    return out[:, :40000]
